# baseline (speedup 1.0000x reference)
import functools

import jax
import jax.numpy as jnp
from jax import lax
from jax.experimental import pallas as pl
from jax.experimental.pallas import tpu as pltpu
from jax.experimental.pallas import tpu_sc as plsc

_DNUMS = lax.GatherDimensionNumbers(
    offset_dims=(), collapsed_slice_dims=(0,), start_index_map=(0,))


def _shuffle(vec, idx):
    return lax.gather(vec, idx.reshape(idx.shape[0], 1), _DNUMS, (1,),
                      mode=lax.GatherScatterMode.PROMISE_IN_BOUNDS)


NC = 2
NS = 16
NW = NC * NS
GRP = 16
NB = 3
LANES = 16


def _fold_body(W1a_hbm, W1b_ref, W2a_hbm, W2b_ref,
               b1a_ref, b1b_ref, b2a_ref, b2b_ref, v_ref, c_ref,
               w1_v, w2_v, sem1, sem2):
    c1 = pltpu.make_async_copy(W1a_hbm, w1_v, sem1)
    c1.start()
    c2 = pltpu.make_async_copy(W2a_hbm, w2_v, sem2)
    c2.start()
    dn = (((0,), (1,)), ((), ()))
    c1.wait()
    v1 = lax.dot_general(W1b_ref[...], w1_v[...], dn,
                         preferred_element_type=jnp.float32)
    c2.wait()
    v2 = lax.dot_general(W2b_ref[...], w2_v[...], dn,
                         preferred_element_type=jnp.float32)
    v_ref[...] = jnp.concatenate([v1, v1 + v2], axis=0)
    c = (jnp.dot(b1a_ref[...], W1b_ref[...])[0, 0] + b1b_ref[0, 0]
         + jnp.dot(b2a_ref[...], W2b_ref[...])[0, 0] + b2b_ref[0, 0])
    c_ref[...] = jnp.full((1, LANES), c, jnp.float32)


@functools.lru_cache(maxsize=None)
def _make_fold(D, H):
    return pl.pallas_call(
        _fold_body,
        in_specs=[
            pl.BlockSpec(memory_space=pl.ANY),
            pl.BlockSpec(memory_space=pltpu.VMEM),
            pl.BlockSpec(memory_space=pl.ANY),
            pl.BlockSpec(memory_space=pltpu.VMEM),
            pl.BlockSpec(memory_space=pltpu.VMEM),
            pl.BlockSpec(memory_space=pltpu.SMEM),
            pl.BlockSpec(memory_space=pltpu.VMEM),
            pl.BlockSpec(memory_space=pltpu.SMEM),
        ],
        out_shape=(
            jax.ShapeDtypeStruct((2, D), jnp.float32),
            jax.ShapeDtypeStruct((1, LANES), jnp.float32),
        ),
        scratch_shapes=[
            pltpu.VMEM((D, H), jnp.float32),
            pltpu.VMEM((D, H), jnp.float32),
            pltpu.SemaphoreType.DMA,
            pltpu.SemaphoreType.DMA,
        ],
    )


TC_RB = 128


@functools.lru_cache(maxsize=None)
def _make_tc(T, D):
    G = T // TC_RB

    def body(x_s, vt_ref, c_s, t1, t2, out_ref, e1b, e2b, sems):
        i = pl.program_id(0)

        def issue(blk, p):
            for r in range(TC_RB):
                rowidx = x_s[blk * TC_RB + r]
                pltpu.make_async_copy(
                    t1.at[pl.ds(rowidx, 1)], e1b.at[p, pl.ds(r, 1)],
                    sems.at[p]).start()
                pltpu.make_async_copy(
                    t2.at[pl.ds(rowidx, 1)], e2b.at[p, pl.ds(r, 1)],
                    sems.at[p]).start()

        @pl.when(i == 0)
        def _():
            issue(0, 0)

        @pl.when(i + 1 < G)
        def _():
            issue(i + 1, (i + 1) % 2)

        p = i % 2
        pltpu.make_async_copy(t1.at[pl.ds(0, TC_RB)], e1b.at[p],
                              sems.at[p]).wait()
        pltpu.make_async_copy(t2.at[pl.ds(0, TC_RB)], e2b.at[p],
                              sems.at[p]).wait()
        r1 = e1b[p]
        r2 = e2b[p]
        o = (jnp.sum(r1 * vt_ref[0:1, :], axis=1, keepdims=True)
             + jnp.sum(r2 * vt_ref[1:2, :], axis=1, keepdims=True))
        out_ref[...] = o + c_s[0, 0]

    return pl.pallas_call(
        body,
        grid=(G,),
        in_specs=[
            pl.BlockSpec(memory_space=pltpu.SMEM),
            pl.BlockSpec((2, D), lambda i: (0, 0)),
            pl.BlockSpec(memory_space=pltpu.SMEM),
            pl.BlockSpec(memory_space=pl.ANY),
            pl.BlockSpec(memory_space=pl.ANY),
        ],
        out_specs=pl.BlockSpec((TC_RB, 1), lambda i: (i, 0)),
        out_shape=jax.ShapeDtypeStruct((T, 1), jnp.float32),
        scratch_shapes=[
            pltpu.VMEM((2, TC_RB, D), jnp.float32),
            pltpu.VMEM((2, TC_RB, D), jnp.float32),
            pltpu.SemaphoreType.DMA((2,)),
        ],
    )


@functools.lru_cache(maxsize=None)
def _make_sc(B, D, skip):
    assert B % NW == 0
    rpw = B // NW
    ng = rpw // GRP
    dc = D // LANES

    mesh = plsc.VectorSubcoreMesh(core_axis_name="c", subcore_axis_name="s",
                                  num_cores=NC, num_subcores=NS)

    def body(x_hbm, t1_hbm, t2_hbm, v_hbm, c_hbm, out_hbm,
             idx_v, v_v, c_v, r1_v, r2_v, out_v, sem0, sem1, sem2):
        wid = lax.axis_index("s") * NC + lax.axis_index("c")
        base = skip + wid * rpw
        pltpu.sync_copy(x_hbm.at[pl.ds(base, rpw)], idx_v)
        pltpu.sync_copy(v_hbm, v_v)
        pltpu.sync_copy(c_hbm, c_v)

        sems = (sem0, sem1, sem2)
        handles = [None] * NB

        def fire(g, b):
            iv = idx_v[pl.ds(g * GRP, GRP)]
            h1 = pltpu.async_copy(t1_hbm.at[iv], r1_v.at[b], sems[b])
            h2 = pltpu.async_copy(t2_hbm.at[iv], r2_v.at[b], sems[b])
            handles[b] = (h1, h2)

        def compute(g, b):
            def jbody(j, accs):
                o = pl.ds(pl.multiple_of(j * LANES, LANES), LANES)
                v1c = v_v[0, o]
                v12c = v_v[1, o]
                return tuple(
                    accs[r] + r1_v[b, r, o] * v1c + r2_v[b, r, o] * v12c
                    for r in range(GRP))

            zero = jnp.zeros((LANES,), jnp.float32)
            accs = lax.fori_loop(0, dc, jbody, (zero,) * GRP)
            lane = lax.iota(jnp.int32, LANES)
            outv = c_v[...]
            for r in range(GRP):
                t = accs[r]
                for sh in (8, 4, 2, 1):
                    t = t + _shuffle(t, jnp.bitwise_xor(lane, sh))
                outv = outv + jnp.where(lane == r, t, 0.0)
            out_v[pl.ds(g * GRP, GRP)] = outv

        for b in range(min(NB, ng)):
            fire(b, b)
        for g in range(ng):
            b = g % NB
            for h in handles[b]:
                h.wait()
            compute(g, b)
            if g + NB < ng:
                fire(g + NB, b)

        pltpu.sync_copy(out_v, out_hbm.at[pl.ds(base - skip, rpw)])

    return pl.kernel(
        body,
        out_type=jax.ShapeDtypeStruct((B,), jnp.float32),
        mesh=mesh,
        scratch_types=[
            pltpu.VMEM((rpw,), jnp.int32),
            pltpu.VMEM((2, D), jnp.float32),
            pltpu.VMEM((LANES,), jnp.float32),
            pltpu.VMEM((NB, GRP, D), jnp.float32),
            pltpu.VMEM((NB, GRP, D), jnp.float32),
            pltpu.VMEM((rpw,), jnp.float32),
            pltpu.SemaphoreType.DMA,
            pltpu.SemaphoreType.DMA,
            pltpu.SemaphoreType.DMA,
        ],
    )


TC_FRAC_ROWS = 1536


def kernel(x, table_1, table_2, W1a, b1a, W1b, b1b, W2a, b2a, W2b, b2b):
    B = x.shape[0]
    D = table_1.shape[1]
    H = W1a.shape[1]
    vt, c = _make_fold(D, H)(W1a, W1b, W2a, W2b,
                             b1a.reshape(1, H), b1b.reshape(1, 1),
                             b2a.reshape(1, H), b2b.reshape(1, 1))
    T = TC_FRAC_ROWS
    out_sc = _make_sc(B - T, D, T)(x, table_1, table_2, vt,
                                   c.reshape(LANES))
    out_tc = _make_tc(T, D)(x, vt, c, table_1, table_2)
    return jnp.concatenate([out_tc, out_sc.reshape(B - T, 1)], axis=0)


# speedup vs baseline: 1.0228x; 1.0228x over previous
import functools

import jax
import jax.numpy as jnp
from jax import lax
from jax.experimental import pallas as pl
from jax.experimental.pallas import tpu as pltpu
from jax.experimental.pallas import tpu_sc as plsc

_DNUMS = lax.GatherDimensionNumbers(
    offset_dims=(), collapsed_slice_dims=(0,), start_index_map=(0,))


def _shuffle(vec, idx):
    return lax.gather(vec, idx.reshape(idx.shape[0], 1), _DNUMS, (1,),
                      mode=lax.GatherScatterMode.PROMISE_IN_BOUNDS)


NC = 2
NS = 16
NW = NC * NS
GRP = 16
NB = 3
LANES = 16


def _fold_body(W1a_ref, W1b_ref, W2a_ref, W2b_ref,
               b1a_ref, b1b_ref, b2a_ref, b2b_ref, v_ref, c_ref):
    dn = (((0,), (1,)), ((), ()))
    v1 = lax.dot_general(W1b_ref[...], W1a_ref[...], dn,
                         preferred_element_type=jnp.float32)
    v2 = lax.dot_general(W2b_ref[...], W2a_ref[...], dn,
                         preferred_element_type=jnp.float32)
    v_ref[...] = jnp.concatenate([v1, v1 + v2], axis=0)
    c = (jnp.dot(b1a_ref[...], W1b_ref[...])[0, 0] + b1b_ref[0, 0]
         + jnp.dot(b2a_ref[...], W2b_ref[...])[0, 0] + b2b_ref[0, 0])
    c_ref[...] = jnp.full((1, LANES), c, jnp.float32)


@functools.lru_cache(maxsize=None)
def _make_fold(D, H):
    return pl.pallas_call(
        _fold_body,
        out_shape=(
            jax.ShapeDtypeStruct((2, D), jnp.float32),
            jax.ShapeDtypeStruct((1, LANES), jnp.float32),
        ),
    )


TC_RB = 128


@functools.lru_cache(maxsize=None)
def _make_tc(T, D):
    G = T // TC_RB

    def body(x_s, vt_ref, c_s, t1, t2, out_ref, e1b, e2b, sems):
        i = pl.program_id(0)

        def issue(blk, p):
            for r in range(TC_RB):
                rowidx = x_s[blk * TC_RB + r]
                pltpu.make_async_copy(
                    t1.at[pl.ds(rowidx, 1)], e1b.at[p, pl.ds(r, 1)],
                    sems.at[p]).start()
                pltpu.make_async_copy(
                    t2.at[pl.ds(rowidx, 1)], e2b.at[p, pl.ds(r, 1)],
                    sems.at[p]).start()

        @pl.when(i == 0)
        def _():
            issue(0, 0)

        @pl.when(i + 1 < G)
        def _():
            issue(i + 1, (i + 1) % 2)

        p = i % 2
        pltpu.make_async_copy(t1.at[pl.ds(0, TC_RB)], e1b.at[p],
                              sems.at[p]).wait()
        pltpu.make_async_copy(t2.at[pl.ds(0, TC_RB)], e2b.at[p],
                              sems.at[p]).wait()
        r1 = e1b[p]
        r2 = e2b[p]
        o = (jnp.sum(r1 * vt_ref[0:1, :], axis=1, keepdims=True)
             + jnp.sum(r2 * vt_ref[1:2, :], axis=1, keepdims=True))
        out_ref[...] = o + c_s[0, 0]

    return pl.pallas_call(
        body,
        grid=(G,),
        in_specs=[
            pl.BlockSpec(memory_space=pltpu.SMEM),
            pl.BlockSpec((2, D), lambda i: (0, 0)),
            pl.BlockSpec(memory_space=pltpu.SMEM),
            pl.BlockSpec(memory_space=pl.ANY),
            pl.BlockSpec(memory_space=pl.ANY),
        ],
        out_specs=pl.BlockSpec((TC_RB, 1), lambda i: (i, 0)),
        out_shape=jax.ShapeDtypeStruct((T, 1), jnp.float32),
        scratch_shapes=[
            pltpu.VMEM((2, TC_RB, D), jnp.float32),
            pltpu.VMEM((2, TC_RB, D), jnp.float32),
            pltpu.SemaphoreType.DMA((2,)),
        ],
    )


@functools.lru_cache(maxsize=None)
def _make_sc(B, D, skip):
    assert B % NW == 0
    rpw = B // NW
    ng = rpw // GRP
    dc = D // LANES

    mesh = plsc.VectorSubcoreMesh(core_axis_name="c", subcore_axis_name="s",
                                  num_cores=NC, num_subcores=NS)

    def body(x_hbm, t1_hbm, t2_hbm, v_hbm, c_hbm, out_hbm,
             idx_v, v_v, c_v, r1_v, r2_v, out_v, sem0, sem1, sem2):
        wid = lax.axis_index("s") * NC + lax.axis_index("c")
        base = skip + wid * rpw
        pltpu.sync_copy(x_hbm.at[pl.ds(base, rpw)], idx_v)
        pltpu.sync_copy(v_hbm, v_v)
        pltpu.sync_copy(c_hbm, c_v)

        sems = (sem0, sem1, sem2)
        handles = [None] * NB

        def fire(g, b):
            iv = idx_v[pl.ds(g * GRP, GRP)]
            h1 = pltpu.async_copy(t1_hbm.at[iv], r1_v.at[b], sems[b])
            h2 = pltpu.async_copy(t2_hbm.at[iv], r2_v.at[b], sems[b])
            handles[b] = (h1, h2)

        def compute(g, b):
            def jbody(j, accs):
                o = pl.ds(pl.multiple_of(j * LANES, LANES), LANES)
                v1c = v_v[0, o]
                v12c = v_v[1, o]
                return tuple(
                    accs[r] + r1_v[b, r, o] * v1c + r2_v[b, r, o] * v12c
                    for r in range(GRP))

            zero = jnp.zeros((LANES,), jnp.float32)
            accs = lax.fori_loop(0, dc, jbody, (zero,) * GRP)
            lane = lax.iota(jnp.int32, LANES)
            outv = c_v[...]
            for r in range(GRP):
                t = accs[r]
                for sh in (8, 4, 2, 1):
                    t = t + _shuffle(t, jnp.bitwise_xor(lane, sh))
                outv = outv + jnp.where(lane == r, t, 0.0)
            out_v[pl.ds(g * GRP, GRP)] = outv

        for b in range(min(NB, ng)):
            fire(b, b)
        for g in range(ng):
            b = g % NB
            for h in handles[b]:
                h.wait()
            compute(g, b)
            if g + NB < ng:
                fire(g + NB, b)

        pltpu.sync_copy(out_v, out_hbm.at[pl.ds(base - skip, rpw)])

    return pl.kernel(
        body,
        out_type=jax.ShapeDtypeStruct((B,), jnp.float32),
        mesh=mesh,
        scratch_types=[
            pltpu.VMEM((rpw,), jnp.int32),
            pltpu.VMEM((2, D), jnp.float32),
            pltpu.VMEM((LANES,), jnp.float32),
            pltpu.VMEM((NB, GRP, D), jnp.float32),
            pltpu.VMEM((NB, GRP, D), jnp.float32),
            pltpu.VMEM((rpw,), jnp.float32),
            pltpu.SemaphoreType.DMA,
            pltpu.SemaphoreType.DMA,
            pltpu.SemaphoreType.DMA,
        ],
    )


TC_FRAC_ROWS = 1536


def kernel(x, table_1, table_2, W1a, b1a, W1b, b1b, W2a, b2a, W2b, b2b):
    B = x.shape[0]
    D = table_1.shape[1]
    H = W1a.shape[1]
    vt, c = _make_fold(D, H)(W1a, W1b, W2a, W2b,
                             b1a.reshape(1, H), b1b.reshape(1, 1),
                             b2a.reshape(1, H), b2b.reshape(1, 1))
    T = TC_FRAC_ROWS
    out_sc = _make_sc(B - T, D, T)(x, table_1, table_2, vt,
                                   c.reshape(LANES))
    out_tc = _make_tc(T, D)(x, vt, c, table_1, table_2)
    return jnp.concatenate([out_tc, out_sc.reshape(B - T, 1)], axis=0)
